# Initial kernel scaffold; baseline (speedup 1.0000x reference)
import jax
import jax.numpy as jnp
from jax.experimental import pallas as pl


def kernel(one_hot_features, w, w0, bias):
    raise NotImplementedError("write your pallas kernel here")


# speedup vs baseline: 2.1391x; 2.1391x over previous
import functools

import jax
import jax.numpy as jnp
from jax import lax
from jax.experimental import pallas as pl
from jax.experimental.pallas import tpu as pltpu
from jax.experimental.pallas import tpu_sc as plsc

B = 16384
F = 26
VOCAB = 1000000
D = 32

NW = 32
SPT = B // NW
CHUNK = 64
NCHUNK = SPT // CHUNK
ROWS_PER_CHUNK = CHUNK * F
GROUP = 128
GROUPS_PER_CHUNK = ROWS_PER_CHUNK // GROUP
IDX_ROWS = SPT * F // GROUP
GPAD = 16

_mesh = plsc.VectorSubcoreMesh(core_axis_name="c", subcore_axis_name="s")


@functools.partial(
    pl.kernel,
    mesh=_mesh,
    out_type=jax.ShapeDtypeStruct((NW, SPT), jnp.float32),
    scratch_types=[
        pltpu.VMEM((IDX_ROWS, GROUP), jnp.int32),
        pltpu.VMEM((ROWS_PER_CHUNK, D), jnp.float32),
        pltpu.VMEM((ROWS_PER_CHUNK, D), jnp.float32),
        pltpu.VMEM((ROWS_PER_CHUNK + GPAD,), jnp.float32),
        pltpu.VMEM((SPT,), jnp.float32),
        pltpu.VMEM((32,), jnp.float32),
        pltpu.VMEM((16,), jnp.float32),
        pltpu.SemaphoreType.DMA,
        pltpu.SemaphoreType.DMA,
    ],
    compiler_params=pltpu.CompilerParams(
        needs_layout_passes=False, use_tc_tiling_on_sc=False),
)
def _fm_sc(idx_hbm, w_hbm, w0_hbm, bias_hbm, out_hbm,
           idx_v, rows0, rows1, g_v, res_v, w0_v, bias_v, sem0, sem1):
    wid = lax.axis_index("s") * 2 + lax.axis_index("c")

    pltpu.sync_copy(idx_hbm.at[wid], idx_v)
    pltpu.sync_copy(w0_hbm, w0_v)
    pltpu.sync_copy(bias_hbm, bias_v)
    bias_vec = bias_v[pl.ds(0, 16)]

    rows_bufs = (rows0, rows1)
    sems = (sem0, sem1)

    def issue_chunk(c, buf, sem):
        copies = []
        for j in range(GROUPS_PER_CHUNK):
            cp = pltpu.make_async_copy(
                w_hbm.at[idx_v.at[c * GROUPS_PER_CHUNK + j]],
                buf.at[pl.ds(j * GROUP, GROUP), :],
                sem,
            )
            cp.start()
            copies.append(cp)
        return copies

    lane = lax.iota(jnp.int32, 16)
    tail_mask = lane < (F - 16)
    zeros = jnp.zeros((16,), jnp.float32)

    def compute_g(c):
        def jbody(j, _):
            def lbody(l, _):
                iv = idx_v[c * GROUPS_PER_CHUNK + j, pl.ds(l * 16, 16)]
                ii = jnp.minimum(jnp.maximum(iv, 0), F - 1)
                vals = plsc.load_gather(w0_v, [ii])
                g_v[pl.ds(j * GROUP + l * 16, 16)] = vals
                return 0
            return lax.fori_loop(0, GROUP // 16, lbody, 0)
        lax.fori_loop(0, GROUPS_PER_CHUNK, jbody, 0)

    def compute_chunk(c, buf):
        base = c * CHUNK

        def gbody(gi, _):
            def sbody(k, resvec):
                off = (gi * 16 + k) * F
                acc0 = zeros
                acc1 = zeros
                q0 = zeros
                q1 = zeros
                for f in range(F):
                    r0 = buf[off + f, pl.ds(0, 16)]
                    r1 = buf[off + f, pl.ds(16, 16)]
                    acc0 = acc0 + r0
                    acc1 = acc1 + r1
                    q0 = q0 + r0 * r0
                    q1 = q1 + r1 * r1
                fm = acc0 * acc0 - q0 + acc1 * acc1 - q1
                ga = g_v[pl.ds(off, 16)]
                gb = jnp.where(tail_mask, g_v[pl.ds(off + 16, 16)], zeros)
                vec = fm * 0.5 + ga + gb
                return jnp.where(lane == k, jnp.sum(vec), resvec)

            resvec = lax.fori_loop(0, 16, sbody, zeros)
            res_v[pl.ds(base + gi * 16, 16)] = resvec + bias_vec
            return 0

        lax.fori_loop(0, CHUNK // 16, gbody, 0)

    pending = issue_chunk(0, rows_bufs[0], sems[0])
    for c in range(NCHUNK):
        nxt = None
        if c + 1 < NCHUNK:
            nxt = issue_chunk(c + 1, rows_bufs[(c + 1) % 2], sems[(c + 1) % 2])
        compute_g(c)
        for cp in pending:
            cp.wait()
        compute_chunk(c, rows_bufs[c % 2])
        pending = nxt

    pltpu.sync_copy(res_v, out_hbm.at[wid])


def kernel(one_hot_features, w, w0, bias):
    idx = one_hot_features.reshape(NW, IDX_ROWS, GROUP).astype(jnp.int32)
    w0p = jnp.pad(w0.reshape(F), (0, 32 - F))
    biasv = jnp.broadcast_to(jnp.reshape(bias, (1,)), (16,)).astype(jnp.float32)
    out = _fm_sc(idx, w, w0p, biasv)
    return out.reshape(B, 1)


# speedup vs baseline: 3.2347x; 1.5122x over previous
import functools

import jax
import jax.numpy as jnp
from jax import lax
from jax.experimental import pallas as pl
from jax.experimental.pallas import tpu as pltpu
from jax.experimental.pallas import tpu_sc as plsc

B = 16384
F = 26
VOCAB = 1000000
D = 32

NW = 32
SPT = B // NW
CHUNK = 64
NCHUNK = SPT // CHUNK
ROWS_PER_CHUNK = CHUNK * F
GROUP = 128
GROUPS_PER_CHUNK = ROWS_PER_CHUNK // GROUP
IDX_ROWS = SPT * F // GROUP
GPAD = 16

VBLK = 4096
NPBLK = (VOCAB + VBLK - 1) // VBLK
VPACK = NPBLK * VBLK


def _pack_body(in_ref, out_ref):
    t = jnp.transpose(in_ref[...])
    parts = [t[1024 * k:1024 * (k + 1), :] for k in range(4)]
    out_ref[...] = jnp.concatenate(parts, axis=1)


def _pack_table(w):
    wT = jnp.swapaxes(w, 0, 1)
    out = pl.pallas_call(
        _pack_body,
        grid=(NPBLK,),
        in_specs=[pl.BlockSpec((D, VBLK), lambda p: (0, p))],
        out_specs=pl.BlockSpec((VBLK * D // 128, 128), lambda p: (p, 0)),
        out_shape=jax.ShapeDtypeStruct((VPACK * D // 128, 128), jnp.float32),
    )(wT)
    return out.reshape(VPACK, D)


_mesh = plsc.VectorSubcoreMesh(core_axis_name="c", subcore_axis_name="s")


@functools.partial(
    pl.kernel,
    mesh=_mesh,
    out_type=jax.ShapeDtypeStruct((NW, SPT), jnp.float32),
    scratch_types=[
        pltpu.VMEM((IDX_ROWS, GROUP), jnp.int32),
        pltpu.VMEM((ROWS_PER_CHUNK, D), jnp.float32),
        pltpu.VMEM((ROWS_PER_CHUNK, D), jnp.float32),
        pltpu.VMEM((2, ROWS_PER_CHUNK + GPAD), jnp.float32),
        pltpu.VMEM((SPT,), jnp.float32),
        pltpu.VMEM((32,), jnp.float32),
        pltpu.VMEM((16,), jnp.float32),
        pltpu.SemaphoreType.DMA,
        pltpu.SemaphoreType.DMA,
    ],
    compiler_params=pltpu.CompilerParams(
        needs_layout_passes=False, use_tc_tiling_on_sc=False),
)
def _fm_sc(idx_hbm, w_hbm, w0_hbm, bias_hbm, out_hbm,
           idx_v, rows0, rows1, g_v, res_v, w0_v, bias_v, sem0, sem1):
    wid = lax.axis_index("s") * 2 + lax.axis_index("c")

    pltpu.sync_copy(idx_hbm.at[wid], idx_v)
    pltpu.sync_copy(w0_hbm, w0_v)
    pltpu.sync_copy(bias_hbm, bias_v)
    bias_vec = bias_v[pl.ds(0, 16)]

    rows_bufs = (rows0, rows1)
    sems = (sem0, sem1)

    lane = lax.iota(jnp.int32, 16)
    tail_mask = lane < (F - 16)
    zeros = jnp.zeros((16,), jnp.float32)

    def prep_chunk(c):
        par = c % 2

        def jbody(j, _):
            row = c * GROUPS_PER_CHUNK + j

            def lbody(l, _):
                iv = idx_v[row, pl.ds(l * 16, 16)]
                ii = jnp.minimum(jnp.maximum(iv, 0), F - 1)
                g_v[par, pl.ds(j * GROUP + l * 16, 16)] = plsc.load_gather(w0_v, [ii])
                vp = ((iv >> 12) << 12) + ((iv & 1023) << 2) + ((iv >> 10) & 3)
                idx_v[row, pl.ds(l * 16, 16)] = vp
                return 0

            return lax.fori_loop(0, GROUP // 16, lbody, 0)

        lax.fori_loop(0, GROUPS_PER_CHUNK, jbody, 0)

    def issue_chunk(c, buf, sem):
        copies = []
        for j in range(GROUPS_PER_CHUNK):
            cp = pltpu.make_async_copy(
                w_hbm.at[idx_v.at[c * GROUPS_PER_CHUNK + j]],
                buf.at[pl.ds(j * GROUP, GROUP), :],
                sem,
            )
            cp.start()
            copies.append(cp)
        return copies

    def compute_chunk(c, buf):
        base = c * CHUNK
        par = c % 2

        def gbody(gi, _):
            def sbody(k, resvec):
                off = (gi * 16 + k) * F
                acc0 = zeros
                acc1 = zeros
                q0 = zeros
                q1 = zeros
                for f in range(F):
                    r0 = buf[off + f, pl.ds(0, 16)]
                    r1 = buf[off + f, pl.ds(16, 16)]
                    acc0 = acc0 + r0
                    acc1 = acc1 + r1
                    q0 = q0 + r0 * r0
                    q1 = q1 + r1 * r1
                fm = acc0 * acc0 - q0 + acc1 * acc1 - q1
                ga = g_v[par, pl.ds(off, 16)]
                gb = jnp.where(tail_mask, g_v[par, pl.ds(off + 16, 16)], zeros)
                vec = fm * 0.5 + ga + gb
                return jnp.where(lane == k, jnp.sum(vec), resvec)

            resvec = lax.fori_loop(0, 16, sbody, zeros)
            res_v[pl.ds(base + gi * 16, 16)] = resvec + bias_vec
            return 0

        lax.fori_loop(0, CHUNK // 16, gbody, 0)

    prep_chunk(0)
    pending = issue_chunk(0, rows_bufs[0], sems[0])
    for c in range(NCHUNK):
        nxt = None
        if c + 1 < NCHUNK:
            prep_chunk(c + 1)
            nxt = issue_chunk(c + 1, rows_bufs[(c + 1) % 2], sems[(c + 1) % 2])
        for cp in pending:
            cp.wait()
        compute_chunk(c, rows_bufs[c % 2])
        pending = nxt

    pltpu.sync_copy(res_v, out_hbm.at[wid])


def kernel(one_hot_features, w, w0, bias):
    idx = one_hot_features.reshape(NW, IDX_ROWS, GROUP).astype(jnp.int32)
    wrows = _pack_table(w)
    w0p = jnp.pad(w0.reshape(F), (0, 32 - F))
    biasv = jnp.broadcast_to(jnp.reshape(bias, (1,)), (16,)).astype(jnp.float32)
    out = _fm_sc(idx, wrows, w0p, biasv)
    return out.reshape(B, 1)


# speedup vs baseline: 7.0017x; 2.1645x over previous
import functools

import jax
import jax.numpy as jnp
from jax import lax
from jax.experimental import pallas as pl
from jax.experimental.pallas import tpu as pltpu
from jax.experimental.pallas import tpu_sc as plsc

B = 16384
F = 26
VOCAB = 1000000
D = 32

NW = 32
SPT = B // NW
CHUNK = 64
NCHUNK = SPT // CHUNK
ROWS_PER_CHUNK = CHUNK * F
GROUP = 128
GROUPS_PER_CHUNK = ROWS_PER_CHUNK // GROUP
IDX_ROWS = SPT * F // GROUP
GPAD = 16

VSH = 14
VBLK = 1 << VSH
NPBLK = (VOCAB + VBLK - 1) // VBLK
VPACK = NPBLK * VBLK
QBLK = VBLK // 4


def _pack_body(in_ref, out_ref):
    for m in range(QBLK // 128):
        s = jnp.concatenate(
            [in_ref[:, QBLK * k + 128 * m:QBLK * k + 128 * (m + 1)]
             for k in range(4)], axis=0)
        out_ref[128 * m:128 * (m + 1), :] = jnp.transpose(s)


def _pack_table(w):
    wT = jnp.swapaxes(w, 0, 1)
    out = pl.pallas_call(
        _pack_body,
        grid=(NPBLK,),
        in_specs=[pl.BlockSpec((D, VBLK), lambda p: (0, p))],
        out_specs=pl.BlockSpec((QBLK, 128), lambda p: (p, 0)),
        out_shape=jax.ShapeDtypeStruct((VPACK * D // 128, 128), jnp.float32),
    )(wT)
    return out.reshape(VPACK, D)


_mesh = plsc.VectorSubcoreMesh(core_axis_name="c", subcore_axis_name="s")


@functools.partial(
    pl.kernel,
    mesh=_mesh,
    out_type=jax.ShapeDtypeStruct((NW, SPT), jnp.float32),
    scratch_types=[
        pltpu.VMEM((IDX_ROWS, GROUP), jnp.int32),
        pltpu.VMEM((ROWS_PER_CHUNK, D), jnp.float32),
        pltpu.VMEM((ROWS_PER_CHUNK, D), jnp.float32),
        pltpu.VMEM((2, ROWS_PER_CHUNK + GPAD), jnp.float32),
        pltpu.VMEM((SPT,), jnp.float32),
        pltpu.VMEM((32,), jnp.float32),
        pltpu.VMEM((16,), jnp.float32),
        pltpu.SemaphoreType.DMA,
        pltpu.SemaphoreType.DMA,
    ],
    compiler_params=pltpu.CompilerParams(
        needs_layout_passes=False, use_tc_tiling_on_sc=False),
)
def _fm_sc(idx_hbm, w_hbm, w0_hbm, bias_hbm, out_hbm,
           idx_v, rows0, rows1, g_v, res_v, w0_v, bias_v, sem0, sem1):
    wid = lax.axis_index("s") * 2 + lax.axis_index("c")

    pltpu.sync_copy(idx_hbm.at[wid], idx_v)
    pltpu.sync_copy(w0_hbm, w0_v)
    pltpu.sync_copy(bias_hbm, bias_v)
    bias_vec = bias_v[pl.ds(0, 16)]

    rows_bufs = (rows0, rows1)
    sems = (sem0, sem1)

    lane = lax.iota(jnp.int32, 16)
    tail_mask = lane < (F - 16)
    zeros = jnp.zeros((16,), jnp.float32)

    def prep_chunk(c):
        par = c % 2

        def jbody(j, _):
            row = c * GROUPS_PER_CHUNK + j

            def lbody(l, _):
                iv = idx_v[row, pl.ds(l * 16, 16)]
                ii = jnp.minimum(jnp.maximum(iv, 0), F - 1)
                g_v[par, pl.ds(j * GROUP + l * 16, 16)] = plsc.load_gather(w0_v, [ii])
                vp = (((iv >> VSH) << VSH) + ((iv & (QBLK - 1)) << 2)
                      + ((iv >> (VSH - 2)) & 3))
                idx_v[row, pl.ds(l * 16, 16)] = vp
                return 0

            return lax.fori_loop(0, GROUP // 16, lbody, 0)

        lax.fori_loop(0, GROUPS_PER_CHUNK, jbody, 0)

    def issue_chunk(c, buf, sem):
        copies = []
        for j in range(GROUPS_PER_CHUNK):
            cp = pltpu.make_async_copy(
                w_hbm.at[idx_v.at[c * GROUPS_PER_CHUNK + j]],
                buf.at[pl.ds(j * GROUP, GROUP), :],
                sem,
            )
            cp.start()
            copies.append(cp)
        return copies

    def compute_chunk(c, buf):
        base = c * CHUNK
        par = c % 2

        def gbody(gi, _):
            def sbody(k, resvec):
                off = (gi * 16 + k) * F
                acc0 = zeros
                acc1 = zeros
                q0 = zeros
                q1 = zeros
                for f in range(F):
                    r0 = buf[off + f, pl.ds(0, 16)]
                    r1 = buf[off + f, pl.ds(16, 16)]
                    acc0 = acc0 + r0
                    acc1 = acc1 + r1
                    q0 = q0 + r0 * r0
                    q1 = q1 + r1 * r1
                fm = acc0 * acc0 - q0 + acc1 * acc1 - q1
                ga = g_v[par, pl.ds(off, 16)]
                gb = jnp.where(tail_mask, g_v[par, pl.ds(off + 16, 16)], zeros)
                vec = fm * 0.5 + ga + gb
                return jnp.where(lane == k, jnp.sum(vec), resvec)

            resvec = lax.fori_loop(0, 16, sbody, zeros)
            res_v[pl.ds(base + gi * 16, 16)] = resvec + bias_vec
            return 0

        lax.fori_loop(0, CHUNK // 16, gbody, 0)

    prep_chunk(0)
    pending = issue_chunk(0, rows_bufs[0], sems[0])
    for c in range(NCHUNK):
        nxt = None
        if c + 1 < NCHUNK:
            prep_chunk(c + 1)
            nxt = issue_chunk(c + 1, rows_bufs[(c + 1) % 2], sems[(c + 1) % 2])
        for cp in pending:
            cp.wait()
        compute_chunk(c, rows_bufs[c % 2])
        pending = nxt

    pltpu.sync_copy(res_v, out_hbm.at[wid])


def kernel(one_hot_features, w, w0, bias):
    idx = one_hot_features.reshape(NW, IDX_ROWS, GROUP).astype(jnp.int32)
    wrows = _pack_table(w)
    w0p = jnp.pad(w0.reshape(F), (0, 32 - F))
    biasv = jnp.broadcast_to(jnp.reshape(bias, (1,)), (16,)).astype(jnp.float32)
    out = _fm_sc(idx, wrows, w0p, biasv)
    return out.reshape(B, 1)
